# baseline (speedup 1.0000x reference)
import functools

import jax
import jax.numpy as jnp
import numpy as np
from jax import lax
from jax.experimental import pallas as pl
from jax.experimental.pallas import tpu as pltpu
from jax.experimental.pallas import tpu_sc as plsc

G = 64
N_NODES = 256
N_TOTAL = G * N_NODES
E = 262144
D = N_NODES * (N_NODES - 1) // 2
HALF_D = D // 2
EPS = 1e-5
ALPHA = 0.5

NC = 2
NS = 16
L = 16

EDGES_PER_WORKER = E // (NC * NS)
EDGE_ROWS_PER_WORKER = EDGES_PER_WORKER // 128

KBLK = 2176
NKBLK = D // KBLK


def _sc_mesh():
    return plsc.VectorSubcoreMesh(
        core_axis_name="c", subcore_axis_name="s", num_cores=NC,
        num_subcores=NS)


_SC_PARAMS = pltpu.CompilerParams(needs_layout_passes=False)


def _zero_vmem(ref, nwords):
    def body(i, _):
        ref[pl.ds(i * L, L)] = jnp.zeros((L,), jnp.float32)
        return 0
    lax.fori_loop(0, nwords // L, body, 0)


def _xw_body(x_ref, w_ref, o_ref):
    xs = x_ref[...]
    w = w_ref[...]
    o_ref[...] = jnp.sum(xs * w, axis=2)


def _compute_xw(x3, w3):
    return pl.pallas_call(
        _xw_body,
        grid=(8,),
        in_specs=[
            pl.BlockSpec((16, 128, 256), lambda t: (t, 0, 0)),
            pl.BlockSpec((1, 1, 256), lambda t: (0, 0, 0)),
        ],
        out_specs=pl.BlockSpec((16, 128), lambda t: (t, 0)),
        out_shape=jax.ShapeDtypeStruct((128, 128), jnp.float32),
    )(x3, w3)


def _deg_body(dst_ref, degp_ref, deg_sh, idx_v, ones_v, zb_v, sem):
    c = lax.axis_index("c")
    s = lax.axis_index("s")
    _zero_vmem(zb_v, 1024)
    pltpu.sync_copy(zb_v, deg_sh.at[pl.ds(s * 1024, 1024)])
    plsc.subcore_barrier()
    row0 = (c * NS + s) * EDGE_ROWS_PER_WORKER
    pltpu.sync_copy(dst_ref.at[pl.ds(row0, EDGE_ROWS_PER_WORKER)], idx_v)
    for j in range(8):
        ones_v[pl.ds(j * L, L)] = jnp.ones((L,), jnp.float32)

    descs = []
    for k in range(EDGE_ROWS_PER_WORKER):
        descs.append(pltpu.async_copy(
            ones_v, deg_sh.at[idx_v.at[k]], sem, add=True))
    for d in descs:
        d.wait()
    plsc.subcore_barrier()
    pltpu.sync_copy(deg_sh.at[pl.ds(s * 1024, 1024)],
                    degp_ref.at[c, pl.ds(s * 1024, 1024)])


def _compute_degp(dst2):
    k = pl.kernel(
        _deg_body,
        out_type=jax.ShapeDtypeStruct((NC, N_TOTAL), jnp.float32),
        mesh=_sc_mesh(),
        compiler_params=_SC_PARAMS,
        scratch_types=[
            pltpu.VMEM_SHARED((N_TOTAL,), jnp.float32),
            pltpu.VMEM((EDGE_ROWS_PER_WORKER, 128), jnp.int32),
            pltpu.VMEM((128,), jnp.float32),
            pltpu.VMEM((1024,), jnp.float32),
            pltpu.SemaphoreType.DMA,
        ],
    )
    return k(dst2)


def _dinv_body(degp_ref, xw_ref, o_ref):
    d = degp_ref[0] + degp_ref[1] + 1.0
    dinv = lax.rsqrt(d)
    o_ref[0] = dinv
    o_ref[1] = dinv * xw_ref[...]


def _compute_dinv(degp3, xw):
    return pl.pallas_call(
        _dinv_body,
        in_specs=[pl.BlockSpec((2, 128, 128), lambda: (0, 0, 0)),
                  pl.BlockSpec((128, 128), lambda: (0, 0))],
        out_specs=pl.BlockSpec((2, 128, 128), lambda: (0, 0, 0)),
        out_shape=jax.ShapeDtypeStruct((2, 128, 128), jnp.float32),
    )(degp3, xw)


def _edge_body(src_ref, dst_ref, dd_ref, x1p_ref,
               x1_sh, dinv_v, dxw_v, src_v, dst_v, val_v, zb_v, sem):
    c = lax.axis_index("c")
    s = lax.axis_index("s")
    _zero_vmem(zb_v, 1024)
    pltpu.sync_copy(zb_v, x1_sh.at[pl.ds(s * 1024, 1024)])
    plsc.subcore_barrier()
    pltpu.sync_copy(dd_ref.at[pl.ds(0, 128)], dinv_v)
    pltpu.sync_copy(dd_ref.at[pl.ds(128, 128)], dxw_v)
    row0 = (c * NS + s) * EDGE_ROWS_PER_WORKER
    pltpu.sync_copy(src_ref.at[pl.ds(row0, EDGE_ROWS_PER_WORKER)], src_v)
    pltpu.sync_copy(dst_ref.at[pl.ds(row0, EDGE_ROWS_PER_WORKER)], dst_v)
    def body(k, _):
        for j in range(8):
            sv = src_v[k, pl.ds(j * L, L)]
            dv = dst_v[k, pl.ds(j * L, L)]
            a = plsc.load_gather(
                dinv_v, [lax.shift_right_logical(dv, 7),
                         lax.bitwise_and(dv, 127)])
            b = plsc.load_gather(
                dxw_v, [lax.shift_right_logical(sv, 7),
                        lax.bitwise_and(sv, 127)])
            val_v[k, pl.ds(j * L, L)] = a * b
        return 0
    lax.fori_loop(0, EDGE_ROWS_PER_WORKER, body, 0)
    descs = []
    for k in range(EDGE_ROWS_PER_WORKER):
        descs.append(pltpu.async_copy(
            val_v.at[k], x1_sh.at[dst_v.at[k]], sem, add=True))
    for d in descs:
        d.wait()
    plsc.subcore_barrier()
    pltpu.sync_copy(x1_sh.at[pl.ds(s * 1024, 1024)],
                    x1p_ref.at[c, pl.ds(s * 1024, 1024)])


def _compute_x1p(src2, dst2, dd):
    k = pl.kernel(
        _edge_body,
        out_type=jax.ShapeDtypeStruct((NC, N_TOTAL), jnp.float32),
        mesh=_sc_mesh(),
        compiler_params=_SC_PARAMS,
        scratch_types=[
            pltpu.VMEM_SHARED((N_TOTAL,), jnp.float32),
            pltpu.VMEM((128, 128), jnp.float32),
            pltpu.VMEM((128, 128), jnp.float32),
            pltpu.VMEM((EDGE_ROWS_PER_WORKER, 128), jnp.int32),
            pltpu.VMEM((EDGE_ROWS_PER_WORKER, 128), jnp.int32),
            pltpu.VMEM((EDGE_ROWS_PER_WORKER, 128), jnp.float32),
            pltpu.VMEM((1024,), jnp.float32),
            pltpu.SemaphoreType.DMA,
        ],
    )
    return k(src2, dst2, dd)


QD = D // 4


def _pack_body(x_ref, tidx_ref, out_ref, tidx_v, xg_v, ob0_v, ob1_v,
               sem0, sem1):
    c = lax.axis_index("c")
    s = lax.axis_index("s")
    w = c * NS + s
    pltpu.sync_copy(tidx_ref, tidx_v)
    descs = [None, None]
    for gi in range(2):
        g = w * 2 + gi
        pltpu.sync_copy(x_ref.at[pl.ds(g * N_NODES, N_NODES)], xg_v)
        for q in range(4):
            b = q % 2
            buf = ob0_v if b == 0 else ob1_v
            sem = sem0 if b == 0 else sem1

            def body(k, _, q=q, buf=buf):
                for u in range(2):
                    off = (k * 2 + u) * L
                    iv = tidx_v[pl.ds(q * QD + off, L)]
                    buf[pl.ds(off, L)] = plsc.load_gather(
                        xg_v, [lax.shift_right_logical(iv, 8),
                               lax.bitwise_and(iv, 255)])
                return 0
            if descs[b] is not None:
                descs[b].wait()
            lax.fori_loop(0, QD // (2 * L), body, 0)
            descs[b] = pltpu.async_copy(
                buf, out_ref.at[pl.ds(g * D + q * QD, QD)], sem)
    for d in descs:
        d.wait()


def _compute_x0p(x, tidx):
    k = pl.kernel(
        _pack_body,
        out_type=jax.ShapeDtypeStruct((G * D,), jnp.float32),
        mesh=_sc_mesh(),
        compiler_params=_SC_PARAMS,
        scratch_types=[
            pltpu.VMEM((D,), jnp.int32),
            pltpu.VMEM((N_NODES, N_NODES), jnp.float32),
            pltpu.VMEM((QD,), jnp.float32),
            pltpu.VMEM((QD,), jnp.float32),
            pltpu.SemaphoreType.DMA,
            pltpu.SemaphoreType.DMA,
        ],
    )
    return k(x, tidx)


def _acc_body(x0_ref, w1_ref, bng_ref, bnb_ref, bnm_ref, bnv_ref,
              aw_ref, ab_ref, o_ref, accA, accB):
    t = pl.program_id(0)

    @pl.when(t == 0)
    def _init():
        accA[...] = jnp.zeros_like(accA)
        accB[...] = jnp.zeros_like(accB)

    sblk = bng_ref[...] * lax.rsqrt(bnv_ref[...] + EPS)
    tblk = bnb_ref[...] - bnm_ref[...] * sblk
    x0b = x0_ref[...] * sblk + tblk
    att = jax.nn.sigmoid(x0b * aw_ref[0, 0] + ab_ref[0, 0])
    accA[...] += lax.dot_general(
        att * x0b, w1_ref[...], (((1,), (1,)), ((), ())),
        preferred_element_type=jnp.float32)
    accB[...] += lax.dot_general(
        1.0 - att, w1_ref[...], (((1,), (1,)), ((), ())),
        preferred_element_type=jnp.float32)

    @pl.when(t == NKBLK - 1)
    def _fin():
        o_ref[0] = accA[...]
        o_ref[1] = accB[...]


def _fin_body(acc_ref, w1h_ref, gp_ref, dd_ref, cb_ref, bhg_ref, bhb_ref,
              bhm_ref, bhv_ref, mb1_ref, m1g_ref, m1b_ref, m1m_ref,
              m1v_ref, w2_ref, mb2_ref, m2g_ref, m2b_ref, m2m_ref, m2v_ref,
              w3_ref, mb3_ref, m3g_ref, m3b_ref, m3m_ref, m3v_ref,
              w4_ref, mb4_ref, o_ref):
    gp = gp_ref[...]
    dd = dd_ref[...]
    x1 = gp[0] + gp[1] + dd[0] * dd[1]
    hr = jnp.sum(x1, axis=1, keepdims=True) * (1.0 / N_NODES)
    hr = hr + cb_ref[0, 0]
    hsc = bhg_ref[0, 0] * lax.rsqrt(bhv_ref[0, 0] + EPS)
    h = (hr - bhm_ref[0, 0]) * hsc + bhb_ref[0, 0]
    z = (acc_ref[0] + (ALPHA * h) * acc_ref[1]
         + h * w1h_ref[...] + mb1_ref[...])
    s1 = m1g_ref[...] * lax.rsqrt(m1v_ref[...] + EPS)
    z = jnp.maximum((z - m1m_ref[...]) * s1 + m1b_ref[...], 0.0)
    z = lax.dot_general(z, w2_ref[...], (((1,), (1,)), ((), ())),
                        preferred_element_type=jnp.float32)
    z = z + mb2_ref[...]
    s2 = m2g_ref[...] * lax.rsqrt(m2v_ref[...] + EPS)
    z = jnp.maximum((z - m2m_ref[...]) * s2 + m2b_ref[...], 0.0)
    z = lax.dot_general(z, w3_ref[...], (((1,), (1,)), ((), ())),
                        preferred_element_type=jnp.float32)
    z = z + mb3_ref[...]
    s3 = m3g_ref[...] * lax.rsqrt(m3v_ref[...] + EPS)
    z = jnp.maximum((z - m3m_ref[...]) * s3 + m3b_ref[...], 0.0)
    z = lax.dot_general(z, w4_ref[...], (((1,), (1,)), ((), ())),
                        preferred_element_type=jnp.float32)
    o_ref[...] = z + mb4_ref[...]


def _row(v):
    return v.reshape(1, -1)


def _compute_acc(x0p, mW1, bn_g, bn_b, bn_m, bn_v, att_W, att_b):
    kblk1 = pl.BlockSpec((1, KBLK), lambda t: (0, t))
    const2 = lambda shape: pl.BlockSpec(shape, lambda t: (0, 0))
    return pl.pallas_call(
        _acc_body,
        grid=(NKBLK,),
        in_specs=[
            pl.BlockSpec((G, KBLK), lambda t: (0, t)),
            pl.BlockSpec((256, KBLK), lambda t: (0, t)),
            kblk1, kblk1, kblk1, kblk1,
            const2((1, 1)), const2((1, 1)),
        ],
        out_specs=pl.BlockSpec((2, G, 256), lambda t: (0, 0, 0)),
        out_shape=jax.ShapeDtypeStruct((2, G, 256), jnp.float32),
        scratch_shapes=[
            pltpu.VMEM((G, 256), jnp.float32),
            pltpu.VMEM((G, 256), jnp.float32),
        ],
        compiler_params=pltpu.CompilerParams(
            dimension_semantics=("arbitrary",)),
    )(x0p, mW1, _row(bn_g), _row(bn_b), _row(bn_m), _row(bn_v),
      att_W, _row(att_b))


def _compute_fin(acc, w1h, gp, dd, conv_b, bnh_g, bnh_b, bnh_m, bnh_v,
                 mb1, m1g, m1b, m1m, m1v, mW2, mb2, m2g, m2b, m2m, m2v,
                 mW3, mb3, m3g, m3b, m3m, m3v, mW4, mb4):
    const2 = lambda shape: pl.BlockSpec(shape, lambda: (0, 0))
    return pl.pallas_call(
        _fin_body,
        in_specs=[
            pl.BlockSpec((2, G, 256), lambda: (0, 0, 0)),
            const2((1, 256)),
            pl.BlockSpec((2, G, 256), lambda: (0, 0, 0)),
            pl.BlockSpec((2, G, 256), lambda: (0, 0, 0)),
            const2((1, 1)),
            const2((1, 1)), const2((1, 1)),
            const2((1, 1)), const2((1, 1)),
            const2((1, 256)),
            const2((1, 256)), const2((1, 256)),
            const2((1, 256)), const2((1, 256)),
            const2((128, 256)), const2((1, 128)),
            const2((1, 128)), const2((1, 128)),
            const2((1, 128)), const2((1, 128)),
            const2((128, 128)), const2((1, 128)),
            const2((1, 128)), const2((1, 128)),
            const2((1, 128)), const2((1, 128)),
            const2((2, 128)), const2((1, 2)),
        ],
        out_specs=pl.BlockSpec((G, 2), lambda: (0, 0)),
        out_shape=jax.ShapeDtypeStruct((G, 2), jnp.float32),
    )(acc, w1h, gp, dd, _row(conv_b), _row(bnh_g), _row(bnh_b),
      _row(bnh_m), _row(bnh_v), _row(mb1), _row(m1g), _row(m1b),
      _row(m1m), _row(m1v), mW2, _row(mb2), _row(m2g), _row(m2b),
      _row(m2m), _row(m2v), mW3, _row(mb3), _row(m3g), _row(m3b),
      _row(m3m), _row(m3v), mW4, _row(mb4))


_IU, _JU = np.triu_indices(N_NODES, k=1)
_TIDX = np.asarray(_IU * N_NODES + _JU, dtype=np.int32)


def kernel(x, edge_index, batch, conv_W, conv_b, bn_g, bn_b, bn_m, bn_v,
           bnh_g, bnh_b, bnh_m, bnh_v, att_W, att_b, mW1, mb1, m1g, m1b,
           m1m, m1v, mW2, mb2, m2g, m2b, m2m, m2v, mW3, mb3, m3g, m3b,
           m3m, m3v, mW4, mb4):
    src2 = edge_index[0].reshape(2048, 128)
    dst2 = edge_index[1].reshape(2048, 128)
    x3 = x.reshape(128, 128, 256)
    w3 = conv_W.reshape(1, 1, 256)
    tidx = jnp.asarray(_TIDX)

    mW1a = lax.slice(mW1, (0, 0), (256, D))
    w1h = lax.slice(mW1, (0, D), (256, D + 1)).reshape(1, 256)

    x0p = _compute_x0p(x, tidx)
    xw = _compute_xw(x3, w3)
    degp = _compute_degp(dst2)
    dd = _compute_dinv(degp.reshape(2, 128, 128), xw)
    x1p = _compute_x1p(src2, dst2, dd.reshape(256, 128))

    acc = _compute_acc(x0p.reshape(G, D), mW1a, bn_g, bn_b, bn_m, bn_v,
                       att_W, att_b)

    return _compute_fin(
        acc, w1h,
        x1p.reshape(NC, G, N_NODES), dd.reshape(NC, G, N_NODES),
        conv_b, bnh_g, bnh_b, bnh_m, bnh_v,
        mb1, m1g, m1b, m1m, m1v, mW2, mb2, m2g, m2b, m2m,
        m2v, mW3, mb3, m3g, m3b, m3m, m3v, mW4, mb4)


# speedup vs baseline: 1.2522x; 1.2522x over previous
import functools

import jax
import jax.numpy as jnp
import numpy as np
from jax import lax
from jax.experimental import pallas as pl
from jax.experimental.pallas import tpu as pltpu
from jax.experimental.pallas import tpu_sc as plsc

G = 64
N_NODES = 256
N_TOTAL = G * N_NODES
E = 262144
D = N_NODES * (N_NODES - 1) // 2
HALF_D = D // 2
EPS = 1e-5
ALPHA = 0.5

NC = 2
NS = 16
L = 16

EDGES_PER_WORKER = E // (NC * NS)
EDGE_ROWS_PER_WORKER = EDGES_PER_WORKER // 128

KBLK = 2176
NKBLK = D // KBLK


def _sc_mesh():
    return plsc.VectorSubcoreMesh(
        core_axis_name="c", subcore_axis_name="s", num_cores=NC,
        num_subcores=NS)


_SC_PARAMS = pltpu.CompilerParams(needs_layout_passes=False)


def _zero_vmem(ref, nwords):
    def body(i, _):
        ref[pl.ds(i * L, L)] = jnp.zeros((L,), jnp.float32)
        return 0
    lax.fori_loop(0, nwords // L, body, 0)


def _xw_body(x_ref, w_ref, o_ref):
    xs = x_ref[...]
    w = w_ref[...]
    o_ref[...] = jnp.sum(xs * w, axis=2)


def _compute_xw(x3, w3):
    return pl.pallas_call(
        _xw_body,
        grid=(8,),
        in_specs=[
            pl.BlockSpec((16, 128, 256), lambda t: (t, 0, 0)),
            pl.BlockSpec((1, 1, 256), lambda t: (0, 0, 0)),
        ],
        out_specs=pl.BlockSpec((16, 128), lambda t: (t, 0)),
        out_shape=jax.ShapeDtypeStruct((128, 128), jnp.float32),
    )(x3, w3)


def _deg_body(dst_ref, degp_ref, deg_sh, idx_v, ones_v, zb_v, sem):
    c = lax.axis_index("c")
    s = lax.axis_index("s")
    _zero_vmem(zb_v, 1024)
    pltpu.sync_copy(zb_v, deg_sh.at[pl.ds(s * 1024, 1024)])
    plsc.subcore_barrier()
    row0 = (c * NS + s) * EDGE_ROWS_PER_WORKER
    pltpu.sync_copy(dst_ref.at[pl.ds(row0, EDGE_ROWS_PER_WORKER)], idx_v)
    for j in range(8):
        ones_v[pl.ds(j * L, L)] = jnp.ones((L,), jnp.float32)

    descs = []
    for k in range(EDGE_ROWS_PER_WORKER):
        descs.append(pltpu.async_copy(
            ones_v, deg_sh.at[idx_v.at[k]], sem, add=True))
    for d in descs:
        d.wait()
    plsc.subcore_barrier()
    pltpu.sync_copy(deg_sh.at[pl.ds(s * 1024, 1024)],
                    degp_ref.at[c, pl.ds(s * 1024, 1024)])


def _compute_degp(dst2):
    k = pl.kernel(
        _deg_body,
        out_type=jax.ShapeDtypeStruct((NC, N_TOTAL), jnp.float32),
        mesh=_sc_mesh(),
        compiler_params=_SC_PARAMS,
        scratch_types=[
            pltpu.VMEM_SHARED((N_TOTAL,), jnp.float32),
            pltpu.VMEM((EDGE_ROWS_PER_WORKER, 128), jnp.int32),
            pltpu.VMEM((128,), jnp.float32),
            pltpu.VMEM((1024,), jnp.float32),
            pltpu.SemaphoreType.DMA,
        ],
    )
    return k(dst2)


def _dinv_body(degp_ref, xw_ref, o_ref):
    d = degp_ref[0] + degp_ref[1] + 1.0
    dinv = lax.rsqrt(d)
    o_ref[0] = dinv
    o_ref[1] = dinv * xw_ref[...]


def _compute_dinv(degp3, xw):
    return pl.pallas_call(
        _dinv_body,
        in_specs=[pl.BlockSpec((2, 128, 128), lambda: (0, 0, 0)),
                  pl.BlockSpec((128, 128), lambda: (0, 0))],
        out_specs=pl.BlockSpec((2, 128, 128), lambda: (0, 0, 0)),
        out_shape=jax.ShapeDtypeStruct((2, 128, 128), jnp.float32),
    )(degp3, xw)


def _edge_body(src_ref, dst_ref, dd_ref, x1p_ref,
               x1_sh, dinv_v, dxw_v, src_v, dst_v, val_v, zb_v, sem):
    c = lax.axis_index("c")
    s = lax.axis_index("s")
    _zero_vmem(zb_v, 1024)
    pltpu.sync_copy(zb_v, x1_sh.at[pl.ds(s * 1024, 1024)])
    plsc.subcore_barrier()
    pltpu.sync_copy(dd_ref.at[pl.ds(0, 128)], dinv_v)
    pltpu.sync_copy(dd_ref.at[pl.ds(128, 128)], dxw_v)
    row0 = (c * NS + s) * EDGE_ROWS_PER_WORKER
    pltpu.sync_copy(src_ref.at[pl.ds(row0, EDGE_ROWS_PER_WORKER)], src_v)
    pltpu.sync_copy(dst_ref.at[pl.ds(row0, EDGE_ROWS_PER_WORKER)], dst_v)
    def body(k, _):
        for j in range(8):
            sv = src_v[k, pl.ds(j * L, L)]
            dv = dst_v[k, pl.ds(j * L, L)]
            a = plsc.load_gather(
                dinv_v, [lax.shift_right_logical(dv, 7),
                         lax.bitwise_and(dv, 127)])
            b = plsc.load_gather(
                dxw_v, [lax.shift_right_logical(sv, 7),
                        lax.bitwise_and(sv, 127)])
            val_v[k, pl.ds(j * L, L)] = a * b
        return 0
    lax.fori_loop(0, EDGE_ROWS_PER_WORKER, body, 0)
    descs = []
    for k in range(EDGE_ROWS_PER_WORKER):
        descs.append(pltpu.async_copy(
            val_v.at[k], x1_sh.at[dst_v.at[k]], sem, add=True))
    for d in descs:
        d.wait()
    plsc.subcore_barrier()
    pltpu.sync_copy(x1_sh.at[pl.ds(s * 1024, 1024)],
                    x1p_ref.at[c, pl.ds(s * 1024, 1024)])


def _compute_x1p(src2, dst2, dd):
    k = pl.kernel(
        _edge_body,
        out_type=jax.ShapeDtypeStruct((NC, N_TOTAL), jnp.float32),
        mesh=_sc_mesh(),
        compiler_params=_SC_PARAMS,
        scratch_types=[
            pltpu.VMEM_SHARED((N_TOTAL,), jnp.float32),
            pltpu.VMEM((128, 128), jnp.float32),
            pltpu.VMEM((128, 128), jnp.float32),
            pltpu.VMEM((EDGE_ROWS_PER_WORKER, 128), jnp.int32),
            pltpu.VMEM((EDGE_ROWS_PER_WORKER, 128), jnp.int32),
            pltpu.VMEM((EDGE_ROWS_PER_WORKER, 128), jnp.float32),
            pltpu.VMEM((1024,), jnp.float32),
            pltpu.SemaphoreType.DMA,
        ],
    )
    return k(src2, dst2, dd)


QD = D // 4


def _pack_body(x_ref, tidx_ref, out_ref, tidx_v, xg_v, ob0_v, ob1_v,
               sem0, sem1):
    c = lax.axis_index("c")
    s = lax.axis_index("s")
    w = c * NS + s
    pltpu.sync_copy(tidx_ref, tidx_v)
    descs = [None, None]
    for gi in range(2):
        g = w * 2 + gi
        pltpu.sync_copy(x_ref.at[pl.ds(g * N_NODES, N_NODES)], xg_v)
        for q in range(4):
            b = q % 2
            buf = ob0_v if b == 0 else ob1_v
            sem = sem0 if b == 0 else sem1

            def body(k, _, q=q, buf=buf):
                for u in range(2):
                    off = (k * 2 + u) * L
                    iv = tidx_v[pl.ds(q * QD + off, L)]
                    buf[pl.ds(off, L)] = plsc.load_gather(
                        xg_v, [lax.shift_right_logical(iv, 8),
                               lax.bitwise_and(iv, 255)])
                return 0
            if descs[b] is not None:
                descs[b].wait()
            lax.fori_loop(0, QD // (2 * L), body, 0)
            descs[b] = pltpu.async_copy(
                buf, out_ref.at[pl.ds(g * D + q * QD, QD)], sem)
    for d in descs:
        d.wait()


def _compute_x0p(x, tidx):
    k = pl.kernel(
        _pack_body,
        out_type=jax.ShapeDtypeStruct((G * D,), jnp.float32),
        mesh=_sc_mesh(),
        compiler_params=_SC_PARAMS,
        scratch_types=[
            pltpu.VMEM((D,), jnp.int32),
            pltpu.VMEM((N_NODES, N_NODES), jnp.float32),
            pltpu.VMEM((QD,), jnp.float32),
            pltpu.VMEM((QD,), jnp.float32),
            pltpu.SemaphoreType.DMA,
            pltpu.SemaphoreType.DMA,
        ],
    )
    return k(x, tidx)


def _acc_body(x0_ref, w1_ref, bng_ref, bnb_ref, bnm_ref, bnv_ref,
              aw_ref, ab_ref, o_ref, accA, accB):
    t = pl.program_id(0)

    @pl.when(t == 0)
    def _init():
        accA[...] = jnp.zeros_like(accA)
        accB[...] = jnp.zeros_like(accB)

    sblk = bng_ref[...] * lax.rsqrt(bnv_ref[...] + EPS)
    tblk = bnb_ref[...] - bnm_ref[...] * sblk
    x0b = x0_ref[...] * sblk + tblk
    att = jax.nn.sigmoid(x0b * aw_ref[0, 0] + ab_ref[0, 0])
    accA[...] += lax.dot_general(
        att * x0b, w1_ref[...], (((1,), (1,)), ((), ())),
        preferred_element_type=jnp.float32)
    accB[...] += lax.dot_general(
        1.0 - att, w1_ref[...], (((1,), (1,)), ((), ())),
        preferred_element_type=jnp.float32)

    @pl.when(t == NKBLK - 1)
    def _fin():
        o_ref[0] = accA[...]
        o_ref[1] = accB[...]


def _fin_body(acc_ref, w1h_ref, gp_ref, dd_ref, cb_ref, bhg_ref, bhb_ref,
              bhm_ref, bhv_ref, mb1_ref, m1g_ref, m1b_ref, m1m_ref,
              m1v_ref, w2_ref, mb2_ref, m2g_ref, m2b_ref, m2m_ref, m2v_ref,
              w3_ref, mb3_ref, m3g_ref, m3b_ref, m3m_ref, m3v_ref,
              w4_ref, mb4_ref, o_ref):
    gp = gp_ref[...]
    dd = dd_ref[...]
    x1 = gp[0] + gp[1] + dd[0] * dd[1]
    hr = jnp.sum(x1, axis=1, keepdims=True) * (1.0 / N_NODES)
    hr = hr + cb_ref[0, 0]
    hsc = bhg_ref[0, 0] * lax.rsqrt(bhv_ref[0, 0] + EPS)
    h = (hr - bhm_ref[0, 0]) * hsc + bhb_ref[0, 0]
    z = (acc_ref[0] + (ALPHA * h) * acc_ref[1]
         + h * w1h_ref[...] + mb1_ref[...])
    s1 = m1g_ref[...] * lax.rsqrt(m1v_ref[...] + EPS)
    z = jnp.maximum((z - m1m_ref[...]) * s1 + m1b_ref[...], 0.0)
    z = lax.dot_general(z, w2_ref[...], (((1,), (1,)), ((), ())),
                        preferred_element_type=jnp.float32)
    z = z + mb2_ref[...]
    s2 = m2g_ref[...] * lax.rsqrt(m2v_ref[...] + EPS)
    z = jnp.maximum((z - m2m_ref[...]) * s2 + m2b_ref[...], 0.0)
    z = lax.dot_general(z, w3_ref[...], (((1,), (1,)), ((), ())),
                        preferred_element_type=jnp.float32)
    z = z + mb3_ref[...]
    s3 = m3g_ref[...] * lax.rsqrt(m3v_ref[...] + EPS)
    z = jnp.maximum((z - m3m_ref[...]) * s3 + m3b_ref[...], 0.0)
    z = lax.dot_general(z, w4_ref[...], (((1,), (1,)), ((), ())),
                        preferred_element_type=jnp.float32)
    o_ref[...] = z + mb4_ref[...]


def _row(v):
    return v.reshape(1, -1)


def _compute_acc(x0p, mW1, bn_g, bn_b, bn_m, bn_v, att_W, att_b):
    kblk1 = pl.BlockSpec((1, KBLK), lambda t: (0, t))
    const2 = lambda shape: pl.BlockSpec(shape, lambda t: (0, 0))
    return pl.pallas_call(
        _acc_body,
        grid=(NKBLK,),
        in_specs=[
            pl.BlockSpec((G, KBLK), lambda t: (0, t)),
            pl.BlockSpec((256, KBLK), lambda t: (0, t)),
            kblk1, kblk1, kblk1, kblk1,
            const2((1, 1)), const2((1, 1)),
        ],
        out_specs=pl.BlockSpec((2, G, 256), lambda t: (0, 0, 0)),
        out_shape=jax.ShapeDtypeStruct((2, G, 256), jnp.float32),
        scratch_shapes=[
            pltpu.VMEM((G, 256), jnp.float32),
            pltpu.VMEM((G, 256), jnp.float32),
        ],
        compiler_params=pltpu.CompilerParams(
            dimension_semantics=("arbitrary",)),
    )(x0p, mW1, _row(bn_g), _row(bn_b), _row(bn_m), _row(bn_v),
      att_W, _row(att_b))


def _compute_fin(acc, w1h, gp, dd, conv_b, bnh_g, bnh_b, bnh_m, bnh_v,
                 mb1, m1g, m1b, m1m, m1v, mW2, mb2, m2g, m2b, m2m, m2v,
                 mW3, mb3, m3g, m3b, m3m, m3v, mW4, mb4):
    const2 = lambda shape: pl.BlockSpec(shape, lambda: (0, 0))
    return pl.pallas_call(
        _fin_body,
        in_specs=[
            pl.BlockSpec((2, G, 256), lambda: (0, 0, 0)),
            const2((1, 256)),
            pl.BlockSpec((2, G, 256), lambda: (0, 0, 0)),
            pl.BlockSpec((2, G, 256), lambda: (0, 0, 0)),
            const2((1, 1)),
            const2((1, 1)), const2((1, 1)),
            const2((1, 1)), const2((1, 1)),
            const2((1, 256)),
            const2((1, 256)), const2((1, 256)),
            const2((1, 256)), const2((1, 256)),
            const2((128, 256)), const2((1, 128)),
            const2((1, 128)), const2((1, 128)),
            const2((1, 128)), const2((1, 128)),
            const2((128, 128)), const2((1, 128)),
            const2((1, 128)), const2((1, 128)),
            const2((1, 128)), const2((1, 128)),
            const2((2, 128)), const2((1, 2)),
        ],
        out_specs=pl.BlockSpec((G, 2), lambda: (0, 0)),
        out_shape=jax.ShapeDtypeStruct((G, 2), jnp.float32),
    )(acc, w1h, gp, dd, _row(conv_b), _row(bnh_g), _row(bnh_b),
      _row(bnh_m), _row(bnh_v), _row(mb1), _row(m1g), _row(m1b),
      _row(m1m), _row(m1v), mW2, _row(mb2), _row(m2g), _row(m2b),
      _row(m2m), _row(m2v), mW3, _row(mb3), _row(m3g), _row(m3b),
      _row(m3m), _row(m3v), mW4, _row(mb4))


_IU, _JU = np.triu_indices(N_NODES, k=1)
_TIDX = np.asarray(_IU * N_NODES + _JU, dtype=np.int32)


def kernel(x, edge_index, batch, conv_W, conv_b, bn_g, bn_b, bn_m, bn_v,
           bnh_g, bnh_b, bnh_m, bnh_v, att_W, att_b, mW1, mb1, m1g, m1b,
           m1m, m1v, mW2, mb2, m2g, m2b, m2m, m2v, mW3, mb3, m3g, m3b,
           m3m, m3v, mW4, mb4):
    src2 = edge_index[0].reshape(2048, 128)
    dst2 = edge_index[1].reshape(2048, 128)
    x3 = x.reshape(128, 128, 256)
    w3 = conv_W.reshape(1, 1, 256)
    tidx = jnp.asarray(_TIDX)

    x0p = _compute_x0p(x, tidx)
    xw = _compute_xw(x3, w3)
    degp = _compute_degp(dst2)
    dd = _compute_dinv(degp.reshape(2, 128, 128), xw)
    x1p = _compute_x1p(src2, dst2, dd.reshape(256, 128))

    acc = _compute_acc(x0p.reshape(G, D), mW1, bn_g, bn_b, bn_m, bn_v,
                       att_W, att_b)

    return _compute_fin(
        acc, mW1[:, D].reshape(1, 256),
        x1p.reshape(NC, G, N_NODES), dd.reshape(NC, G, N_NODES),
        conv_b, bnh_g, bnh_b, bnh_m, bnh_v,
        mb1, m1g, m1b, m1m, m1v, mW2, mb2, m2g, m2b, m2m,
        m2v, mW3, mb3, m3g, m3b, m3m, m3v, mW4, mb4)
